# Initial kernel scaffold; baseline (speedup 1.0000x reference)
import jax
import jax.numpy as jnp
from jax.experimental import pallas as pl


def kernel(x):
    raise NotImplementedError("write your pallas kernel here")


# speedup vs baseline: 80.8249x; 80.8249x over previous
import functools

import jax
import jax.numpy as jnp
from jax.experimental import pallas as pl
from jax.experimental.pallas import tpu as pltpu

_TOPK_FRAC = 0.25


def _select_body(x_ref, o_ref, *, k):
    xv = x_ref[0]
    bits = jax.lax.bitcast_convert_type(xv, jnp.int32) & jnp.int32(0x7FFFFFFF)
    hi = jnp.max(bits, axis=0, keepdims=True)
    lo = jnp.zeros_like(hi)

    def step(_, carry):
        lo, hi = carry
        mid = lo + (hi - lo + 1) // 2
        cnt = jnp.sum((bits >= mid).astype(jnp.int32), axis=0, keepdims=True)
        ge = cnt >= k
        return jnp.where(ge, mid, lo), jnp.where(ge, hi, mid - 1)

    lo, hi = jax.lax.fori_loop(0, 31, step, (lo, hi))
    keep = bits >= lo
    o_ref[0] = jnp.where(keep, xv, jnp.float32(0.0))


def kernel(x):
    n, hw1, d = x.shape
    k = max(1, int(hw1 * _TOPK_FRAC))
    cblk = min(d, 256)
    grid = (n, d // cblk)
    return pl.pallas_call(
        functools.partial(_select_body, k=k),
        grid=grid,
        in_specs=[pl.BlockSpec((1, hw1, cblk), lambda i, j: (i, 0, j))],
        out_specs=pl.BlockSpec((1, hw1, cblk), lambda i, j: (i, 0, j)),
        out_shape=jax.ShapeDtypeStruct(x.shape, x.dtype),
        compiler_params=pltpu.CompilerParams(
            dimension_semantics=("parallel", "parallel"),
        ),
    )(x)
